# Initial kernel scaffold; baseline (speedup 1.0000x reference)
import jax
import jax.numpy as jnp
from jax.experimental import pallas as pl


def kernel(arc_scores, wers, arc_to_path, path_to_utt):
    raise NotImplementedError("write your pallas kernel here")


# speedup vs baseline: 17.2017x; 17.2017x over previous
import functools

import jax
import jax.numpy as jnp
from jax import lax
from jax.experimental import pallas as pl
from jax.experimental.pallas import tpu as pltpu
from jax.experimental.pallas import tpu_sc as plsc

NUM_UTTS = 16
NUM_PATHS = 3200
TOTAL_ARCS = 1638400

NC = 2
NS = 16
L = 16
NW = NC * NS
ARCS_PER_WORKER = TOTAL_ARCS // NW
VECS_PER_WORKER = ARCS_PER_WORKER // L


def _sc_segment_sum_body(scores_hbm, ids_hbm, out_hbm, scores_v, ids_v, acc_v):
    wid = lax.axis_index("s") * NC + lax.axis_index("c")
    base = wid * ARCS_PER_WORKER

    pltpu.sync_copy(scores_hbm.at[pl.ds(base, ARCS_PER_WORKER)], scores_v)
    pltpu.sync_copy(ids_hbm.at[pl.ds(base, ARCS_PER_WORKER)], ids_v)

    def zero_body(i, _):
        acc_v[pl.ds(i * L, L)] = jnp.zeros((L,), jnp.float32)
        return _

    lax.fori_loop(0, NUM_PATHS // L, zero_body, None)

    def scatter_body(i, _):
        idx = ids_v[pl.ds(i * L, L)]
        val = scores_v[pl.ds(i * L, L)]
        plsc.addupdate_scatter(acc_v, [idx], val)
        return _

    lax.fori_loop(0, VECS_PER_WORKER, scatter_body, None)

    pltpu.sync_copy(acc_v, out_hbm.at[wid])


@jax.jit
def _sc_segment_sum(arc_scores, arc_to_path):
    mesh = plsc.VectorSubcoreMesh(
        core_axis_name="c", subcore_axis_name="s", num_cores=NC, num_subcores=NS
    )
    return pl.kernel(
        _sc_segment_sum_body,
        out_type=jax.ShapeDtypeStruct((NW, NUM_PATHS), jnp.float32),
        mesh=mesh,
        scratch_types=[
            pltpu.VMEM((ARCS_PER_WORKER,), jnp.float32),
            pltpu.VMEM((ARCS_PER_WORKER,), jnp.int32),
            pltpu.VMEM((NUM_PATHS,), jnp.float32),
        ],
        compiler_params=pltpu.CompilerParams(needs_layout_passes=False),
        name="mwer_segment_sum_sc",
    )(arc_scores, arc_to_path)


def _tc_tail_body(partials_ref, wers_ref, utt_ref, out_ref):
    path_logp = jnp.sum(partials_ref[...], axis=0, keepdims=True)
    prob = jnp.exp(path_logp)
    pw = prob * wers_ref[...]
    utt = utt_ref[...]
    loss = jnp.float32(0.0)
    for u in range(NUM_UTTS):
        m = utt == u
        den = jnp.sum(jnp.where(m, prob, 0.0))
        num = jnp.sum(jnp.where(m, pw, 0.0))
        loss = loss + jnp.where(den > 0, num / den, 0.0)
    out_ref[...] = jnp.broadcast_to(loss, (1, 1))


@jax.jit
def _tc_tail(partials, wers, path_to_utt):
    return pl.pallas_call(
        _tc_tail_body,
        out_shape=jax.ShapeDtypeStruct((1, 1), jnp.float32),
    )(partials, wers.reshape(1, NUM_PATHS), path_to_utt.reshape(1, NUM_PATHS))


def kernel(arc_scores, wers, arc_to_path, path_to_utt):
    partials = _sc_segment_sum(arc_scores, arc_to_path)
    loss = _tc_tail(partials, wers, path_to_utt)
    return loss[0, 0]


# speedup vs baseline: 17.2937x; 1.0054x over previous
import functools

import jax
import jax.numpy as jnp
from jax import lax
from jax.experimental import pallas as pl
from jax.experimental.pallas import tpu as pltpu
from jax.experimental.pallas import tpu_sc as plsc

NUM_UTTS = 16
NUM_PATHS = 3200
TOTAL_ARCS = 1638400

NC = 2
NS = 16
L = 16
NW = NC * NS
ARCS_PER_WORKER = TOTAL_ARCS // NW
VECS_PER_WORKER = ARCS_PER_WORKER // L


def _sc_segment_sum_body(scores_hbm, ids_hbm, out_hbm, scores_v, ids_v, acc_v):
    wid = lax.axis_index("s") * NC + lax.axis_index("c")
    base = wid * ARCS_PER_WORKER

    pltpu.sync_copy(scores_hbm.at[pl.ds(base, ARCS_PER_WORKER)], scores_v)
    pltpu.sync_copy(ids_hbm.at[pl.ds(base, ARCS_PER_WORKER)], ids_v)

    def zero_body(i, _):
        acc_v[pl.ds(i * L, L)] = jnp.zeros((L,), jnp.float32)
        return _

    lax.fori_loop(0, NUM_PATHS // L, zero_body, None, unroll=8)

    def scatter_body(i, _):
        idx = ids_v[pl.ds(i * L, L)]
        val = scores_v[pl.ds(i * L, L)]
        plsc.addupdate_scatter(acc_v, [idx], val)
        return _

    lax.fori_loop(0, VECS_PER_WORKER, scatter_body, None, unroll=8)

    pltpu.sync_copy(acc_v, out_hbm.at[wid])


@jax.jit
def _sc_segment_sum(arc_scores, arc_to_path):
    mesh = plsc.VectorSubcoreMesh(
        core_axis_name="c", subcore_axis_name="s", num_cores=NC, num_subcores=NS
    )
    return pl.kernel(
        _sc_segment_sum_body,
        out_type=jax.ShapeDtypeStruct((NW, NUM_PATHS), jnp.float32),
        mesh=mesh,
        scratch_types=[
            pltpu.VMEM((ARCS_PER_WORKER,), jnp.float32),
            pltpu.VMEM((ARCS_PER_WORKER,), jnp.int32),
            pltpu.VMEM((NUM_PATHS,), jnp.float32),
        ],
        compiler_params=pltpu.CompilerParams(needs_layout_passes=False),
        name="mwer_segment_sum_sc",
    )(arc_scores, arc_to_path)


def _tc_tail_body(partials_ref, wers_ref, utt_ref, out_ref):
    path_logp = jnp.sum(partials_ref[...], axis=0, keepdims=True)
    prob = jnp.exp(path_logp)
    pw = prob * wers_ref[...]
    utt = utt_ref[...]
    loss = jnp.float32(0.0)
    for u in range(NUM_UTTS):
        m = utt == u
        den = jnp.sum(jnp.where(m, prob, 0.0))
        num = jnp.sum(jnp.where(m, pw, 0.0))
        loss = loss + jnp.where(den > 0, num / den, 0.0)
    out_ref[...] = jnp.broadcast_to(loss, (1, 1))


@jax.jit
def _tc_tail(partials, wers, path_to_utt):
    return pl.pallas_call(
        _tc_tail_body,
        out_shape=jax.ShapeDtypeStruct((1, 1), jnp.float32),
    )(partials, wers.reshape(1, NUM_PATHS), path_to_utt.reshape(1, NUM_PATHS))


def kernel(arc_scores, wers, arc_to_path, path_to_utt):
    partials = _sc_segment_sum(arc_scores, arc_to_path)
    loss = _tc_tail(partials, wers, path_to_utt)
    return loss[0, 0]


# speedup vs baseline: 18.5046x; 1.0700x over previous
import functools

import jax
import jax.numpy as jnp
from jax import lax
from jax.experimental import pallas as pl
from jax.experimental.pallas import tpu as pltpu
from jax.experimental.pallas import tpu_sc as plsc

NUM_UTTS = 16
NUM_PATHS = 3200
TOTAL_ARCS = 1638400

NC = 2
NS = 16
L = 16
NW = NC * NS
ARCS_PER_WORKER = TOTAL_ARCS // NW
VECS_PER_WORKER = ARCS_PER_WORKER // L


def _sc_segment_sum_body(scores_hbm, ids_hbm, out_hbm, scores_v, ids_v, acc_v):
    wid = lax.axis_index("s") * NC + lax.axis_index("c")
    base = wid * ARCS_PER_WORKER

    pltpu.sync_copy(scores_hbm.at[pl.ds(base, ARCS_PER_WORKER)], scores_v)
    pltpu.sync_copy(ids_hbm.at[pl.ds(base, ARCS_PER_WORKER)], ids_v)

    def zero_body(i, _):
        acc_v[pl.ds(i * L, L)] = jnp.zeros((L,), jnp.float32)
        return _

    lax.fori_loop(0, NUM_PATHS // L, zero_body, None, unroll=8)

    @plsc.parallel_loop(0, VECS_PER_WORKER, unroll=8)
    def scatter_body(i):
        idx = ids_v[pl.ds(i * L, L)]
        val = scores_v[pl.ds(i * L, L)]
        plsc.addupdate_scatter(acc_v, [idx], val)

    pltpu.sync_copy(acc_v, out_hbm.at[wid])


@jax.jit
def _sc_segment_sum(arc_scores, arc_to_path):
    mesh = plsc.VectorSubcoreMesh(
        core_axis_name="c", subcore_axis_name="s", num_cores=NC, num_subcores=NS
    )
    return pl.kernel(
        _sc_segment_sum_body,
        out_type=jax.ShapeDtypeStruct((NW, NUM_PATHS), jnp.float32),
        mesh=mesh,
        scratch_types=[
            pltpu.VMEM((ARCS_PER_WORKER,), jnp.float32),
            pltpu.VMEM((ARCS_PER_WORKER,), jnp.int32),
            pltpu.VMEM((NUM_PATHS,), jnp.float32),
        ],
        compiler_params=pltpu.CompilerParams(needs_layout_passes=False),
        name="mwer_segment_sum_sc",
    )(arc_scores, arc_to_path)


def _tc_tail_body(partials_ref, wers_ref, utt_ref, out_ref):
    path_logp = jnp.sum(partials_ref[...], axis=0, keepdims=True)
    prob = jnp.exp(path_logp)
    pw = prob * wers_ref[...]
    utt = utt_ref[...]
    loss = jnp.float32(0.0)
    for u in range(NUM_UTTS):
        m = utt == u
        den = jnp.sum(jnp.where(m, prob, 0.0))
        num = jnp.sum(jnp.where(m, pw, 0.0))
        loss = loss + jnp.where(den > 0, num / den, 0.0)
    out_ref[...] = jnp.broadcast_to(loss, (1, 1))


@jax.jit
def _tc_tail(partials, wers, path_to_utt):
    return pl.pallas_call(
        _tc_tail_body,
        out_shape=jax.ShapeDtypeStruct((1, 1), jnp.float32),
    )(partials, wers.reshape(1, NUM_PATHS), path_to_utt.reshape(1, NUM_PATHS))


def kernel(arc_scores, wers, arc_to_path, path_to_utt):
    partials = _sc_segment_sum(arc_scores, arc_to_path)
    loss = _tc_tail(partials, wers, path_to_utt)
    return loss[0, 0]


# speedup vs baseline: 23.4781x; 1.2688x over previous
import functools

import jax
import jax.numpy as jnp
from jax import lax
from jax.experimental import pallas as pl
from jax.experimental.pallas import tpu as pltpu
from jax.experimental.pallas import tpu_sc as plsc

NUM_UTTS = 16
NUM_PATHS = 3200
TOTAL_ARCS = 1638400

NC = 2
NS = 16
L = 16
NW = NC * NS
ARCS_PER_WORKER = TOTAL_ARCS // NW
VECS_PER_WORKER = ARCS_PER_WORKER // L


CHUNK = 12800
NCHUNKS = ARCS_PER_WORKER // CHUNK


def _sc_segment_sum_body(scores_hbm, ids_hbm, out_hbm, scores_v, ids_v, acc_v, row_v):
    wid = lax.axis_index("s") * NC + lax.axis_index("c")
    base = wid * ARCS_PER_WORKER
    lane = jax.lax.iota(jnp.int32, L)
    zeros = jnp.zeros((L,), jnp.float32)

    for r in range(L):

        @plsc.parallel_loop(0, NUM_PATHS // L, unroll=8)
        def zero_body(i):
            acc_v[r, pl.ds(i * L, L)] = zeros

    for k in range(NCHUNKS):
        pltpu.sync_copy(scores_hbm.at[pl.ds(base + k * CHUNK, CHUNK)], scores_v)
        pltpu.sync_copy(ids_hbm.at[pl.ds(base + k * CHUNK, CHUNK)], ids_v)

        @plsc.parallel_loop(0, CHUNK // L, unroll=8)
        def scatter_body(i):
            idx = ids_v[pl.ds(i * L, L)]
            val = scores_v[pl.ds(i * L, L)]
            plsc.addupdate_scatter(acc_v, [lane, idx], val)

    @plsc.parallel_loop(0, NUM_PATHS // L, unroll=4)
    def reduce_body(i):
        s = acc_v[0, pl.ds(i * L, L)]
        for r in range(1, L):
            s = s + acc_v[r, pl.ds(i * L, L)]
        row_v[pl.ds(i * L, L)] = s

    pltpu.sync_copy(row_v, out_hbm.at[wid])


@jax.jit
def _sc_segment_sum(arc_scores, arc_to_path):
    mesh = plsc.VectorSubcoreMesh(
        core_axis_name="c", subcore_axis_name="s", num_cores=NC, num_subcores=NS
    )
    return pl.kernel(
        _sc_segment_sum_body,
        out_type=jax.ShapeDtypeStruct((NW, NUM_PATHS), jnp.float32),
        mesh=mesh,
        scratch_types=[
            pltpu.VMEM((CHUNK,), jnp.float32),
            pltpu.VMEM((CHUNK,), jnp.int32),
            pltpu.VMEM((L, NUM_PATHS), jnp.float32),
            pltpu.VMEM((NUM_PATHS,), jnp.float32),
        ],
        compiler_params=pltpu.CompilerParams(needs_layout_passes=False),
        name="mwer_segment_sum_sc",
    )(arc_scores, arc_to_path)


def _tc_tail_body(partials_ref, wers_ref, utt_ref, out_ref):
    path_logp = jnp.sum(partials_ref[...], axis=0, keepdims=True)
    prob = jnp.exp(path_logp)
    pw = prob * wers_ref[...]
    utt = utt_ref[...]
    loss = jnp.float32(0.0)
    for u in range(NUM_UTTS):
        m = utt == u
        den = jnp.sum(jnp.where(m, prob, 0.0))
        num = jnp.sum(jnp.where(m, pw, 0.0))
        loss = loss + jnp.where(den > 0, num / den, 0.0)
    out_ref[...] = jnp.broadcast_to(loss, (1, 1))


@jax.jit
def _tc_tail(partials, wers, path_to_utt):
    return pl.pallas_call(
        _tc_tail_body,
        out_shape=jax.ShapeDtypeStruct((1, 1), jnp.float32),
    )(partials, wers.reshape(1, NUM_PATHS), path_to_utt.reshape(1, NUM_PATHS))


def kernel(arc_scores, wers, arc_to_path, path_to_utt):
    partials = _sc_segment_sum(arc_scores, arc_to_path)
    loss = _tc_tail(partials, wers, path_to_utt)
    return loss[0, 0]


# speedup vs baseline: 25.5134x; 1.0867x over previous
import functools

import jax
import jax.numpy as jnp
from jax import lax
from jax.experimental import pallas as pl
from jax.experimental.pallas import tpu as pltpu
from jax.experimental.pallas import tpu_sc as plsc

NUM_UTTS = 16
NUM_PATHS = 3200
TOTAL_ARCS = 1638400

NC = 2
NS = 16
L = 16
NW = NC * NS
ARCS_PER_WORKER = TOTAL_ARCS // NW
VECS_PER_WORKER = ARCS_PER_WORKER // L


CHUNK = 12800
NCHUNKS = ARCS_PER_WORKER // CHUNK


def _sc_segment_sum_body(
    scores_hbm, ids_hbm, out_hbm, scores_v, ids_v, acc_v, row_v, sem0, sem1
):
    wid = lax.axis_index("s") * NC + lax.axis_index("c")
    base = wid * ARCS_PER_WORKER
    lane = jax.lax.iota(jnp.int32, L)
    zeros = jnp.zeros((L,), jnp.float32)
    sems = (sem0, sem1)

    def start(k):
        b = k % 2
        sl = pl.ds(base + k * CHUNK, CHUNK)
        return (
            pltpu.async_copy(scores_hbm.at[sl], scores_v.at[b], sems[b]),
            pltpu.async_copy(ids_hbm.at[sl], ids_v.at[b], sems[b]),
        )

    pending = start(0)

    for r in range(L):

        @plsc.parallel_loop(0, NUM_PATHS // L, unroll=8)
        def zero_body(i):
            acc_v[r, pl.ds(i * L, L)] = zeros

    for k in range(NCHUNKS):
        b = k % 2
        for h in pending:
            h.wait()
        if k + 1 < NCHUNKS:
            pending = start(k + 1)

        @plsc.parallel_loop(0, CHUNK // L, unroll=8)
        def scatter_body(i):
            idx = ids_v[b, pl.ds(i * L, L)]
            val = scores_v[b, pl.ds(i * L, L)]
            plsc.addupdate_scatter(acc_v, [lane, idx], val)

    @plsc.parallel_loop(0, NUM_PATHS // L, unroll=4)
    def reduce_body(i):
        s = acc_v[0, pl.ds(i * L, L)]
        for r in range(1, L):
            s = s + acc_v[r, pl.ds(i * L, L)]
        row_v[pl.ds(i * L, L)] = s

    pltpu.sync_copy(row_v, out_hbm.at[wid])


@jax.jit
def _sc_segment_sum(arc_scores, arc_to_path):
    mesh = plsc.VectorSubcoreMesh(
        core_axis_name="c", subcore_axis_name="s", num_cores=NC, num_subcores=NS
    )
    return pl.kernel(
        _sc_segment_sum_body,
        out_type=jax.ShapeDtypeStruct((NW, NUM_PATHS), jnp.float32),
        mesh=mesh,
        scratch_types=[
            pltpu.VMEM((2, CHUNK), jnp.float32),
            pltpu.VMEM((2, CHUNK), jnp.int32),
            pltpu.VMEM((L, NUM_PATHS), jnp.float32),
            pltpu.VMEM((NUM_PATHS,), jnp.float32),
            pltpu.SemaphoreType.DMA,
            pltpu.SemaphoreType.DMA,
        ],
        compiler_params=pltpu.CompilerParams(needs_layout_passes=False),
        name="mwer_segment_sum_sc",
    )(arc_scores, arc_to_path)


def _tc_tail_body(partials_ref, wers_ref, utt_ref, out_ref):
    path_logp = jnp.sum(partials_ref[...], axis=0, keepdims=True)
    prob = jnp.exp(path_logp)
    pw = prob * wers_ref[...]
    utt = utt_ref[...]
    loss = jnp.float32(0.0)
    for u in range(NUM_UTTS):
        m = utt == u
        den = jnp.sum(jnp.where(m, prob, 0.0))
        num = jnp.sum(jnp.where(m, pw, 0.0))
        loss = loss + jnp.where(den > 0, num / den, 0.0)
    out_ref[...] = jnp.broadcast_to(loss, (1, 1))


@jax.jit
def _tc_tail(partials, wers, path_to_utt):
    return pl.pallas_call(
        _tc_tail_body,
        out_shape=jax.ShapeDtypeStruct((1, 1), jnp.float32),
    )(partials, wers.reshape(1, NUM_PATHS), path_to_utt.reshape(1, NUM_PATHS))


def kernel(arc_scores, wers, arc_to_path, path_to_utt):
    partials = _sc_segment_sum(arc_scores, arc_to_path)
    loss = _tc_tail(partials, wers, path_to_utt)
    return loss[0, 0]
